# baseline (speedup 1.0000x reference)
import functools

import jax
import jax.numpy as jnp
from jax import lax
from jax.experimental import pallas as pl
from jax.experimental.pallas import tpu as pltpu
from jax.experimental.pallas import tpu_sc as plsc

N = 10000
E = 320000
D = 128
CH = 56


def _sc_segment_sums(WA0, WA1, src, dst):
  info = plsc.get_sparse_core_info()
  ns = info.num_subcores
  e_per_tile = E // ns
  n_main = e_per_tile // CH
  tail = e_per_tile - n_main * CH
  zrows = 40
  n_rowch = N // zrows
  max_per_tile = -(-n_rowch // ns)

  NSETS = 6
  GA = 3
  SLAG = 2
  IA = 4
  n_groups = n_main // NSETS

  mesh = plsc.VectorSubcoreMesh(core_axis_name="c", subcore_axis_name="s")

  scratch = []
  for _ in range(NSETS):
    scratch += [
        pltpu.VMEM((CH,), jnp.int32),
        pltpu.VMEM((CH,), jnp.int32),
        pltpu.VMEM((CH, D), jnp.float32),
        pltpu.SemaphoreType.DMA,
        pltpu.SemaphoreType.DMA,
        pltpu.SemaphoreType.DMA,
    ]
  scratch += [
      pltpu.VMEM((zrows, D), jnp.float32),
      pltpu.VMEM_SHARED((N, D), jnp.float32),
  ]
  if tail:
    scratch += [
        pltpu.VMEM((tail,), jnp.int32),
        pltpu.VMEM((tail,), jnp.int32),
    ]

  @functools.partial(
      pl.kernel,
      out_type=[
          jax.ShapeDtypeStruct((N, D), jnp.float32),
          jax.ShapeDtypeStruct((N, D), jnp.float32),
      ],
      mesh=mesh,
      scratch_types=scratch,
  )
  def body(wa0_hbm, wa1_hbm, src_hbm, dst_hbm, out0_hbm, out1_hbm, *bufs):
    sets = [bufs[6 * m:6 * m + 6] for m in range(NSETS)]
    zbuf = bufs[6 * NSETS]
    acc_sh = bufs[6 * NSETS + 1]
    tailbufs = bufs[6 * NSETS + 2:]
    cid = lax.axis_index("c")
    sid = lax.axis_index("s")
    ebase = pl.multiple_of(sid * e_per_tile, 8)

    z16 = jnp.zeros((16,), jnp.float32)

    def zrow(r, carry):
      for l in range(D // 16):
        zbuf[r, pl.ds(l * 16, 16)] = z16
      return carry

    lax.fori_loop(0, zrows, zrow, 0)
    for k in range(max_per_tile):
      j = k * ns + sid

      @pl.when(j < n_rowch)
      def _():
        pltpu.sync_copy(zbuf, acc_sh.at[pl.ds(pl.multiple_of(j * zrows, 8),
                                              zrows)])

    def idx_start(j, s):
      srcv, dstv, _, isem, _, _ = s
      base = pl.multiple_of(ebase + j * CH, 8)
      pltpu.async_copy(src_hbm.at[pl.ds(base, CH)], srcv, isem)
      pltpu.async_copy(dst_hbm.at[pl.ds(base, CH)], dstv, isem)

    def idx_wait(j, s):
      srcv, dstv, _, isem, _, _ = s
      base = pl.multiple_of(ebase + j * CH, 8)
      pltpu.make_async_copy(src_hbm.at[pl.ds(base, CH)], srcv, isem).wait()
      pltpu.make_async_copy(dst_hbm.at[pl.ds(base, CH)], dstv, isem).wait()

    for jj in range(IA):
      idx_start(jj, sets[jj])
    plsc.subcore_barrier()

    def accumulate(table_hbm):
      def gather_start(s):
        srcv, _, rows, _, gsem, _ = s
        pltpu.async_copy(table_hbm.at[srcv], rows, gsem)

      def gather_wait(s):
        srcv, _, rows, _, gsem, _ = s
        pltpu.make_async_copy(table_hbm.at[srcv], rows, gsem).wait()

      def scatter_start(s):
        _, dstv, rows, _, _, ssem = s
        pltpu.async_copy(rows, acc_sh.at[dstv], ssem, add=True)

      def scatter_wait(s):
        _, dstv, rows, _, _, ssem = s
        pltpu.make_async_copy(rows, acc_sh.at[dstv], ssem).wait()

      for jj in range(GA):
        idx_wait(jj, sets[jj])
        gather_start(sets[jj])

      def group(i, carry):
        for m in range(NSETS):
          j = i * NSETS + m
          X = sets[m]
          W = sets[(m + GA) % NSETS]
          V = sets[(m + IA) % NSETS]
          gather_wait(X)
          scatter_start(X)

          def next_gather(j=j, W=W):
            idx_wait(j + GA, W)
            gather_start(W)

          tg = (n_main - 1 - GA - m) // NSETS
          if tg >= n_groups - 1:
            next_gather()
          else:
            @pl.when(i <= tg)
            def _():
              next_gather()

          if m < SLAG:
            @pl.when(i > 0)
            def _():
              scatter_wait(V)
          else:
            scatter_wait(V)

          def next_idx(j=j, V=V):
            idx_start(j + IA, V)

          ti = (n_main - 1 - IA - m) // NSETS
          if ti >= n_groups - 1:
            next_idx()
          else:
            @pl.when(i <= ti)
            def _():
              next_idx()
        return carry

      lax.fori_loop(0, n_groups, group, 0)

      for j in range(n_groups * NSETS, n_main):
        X = sets[j % NSETS]
        gather_wait(X)
        scatter_start(X)
        scatter_wait(sets[(j - SLAG) % NSETS])

      if tail:
        srcv_t, dstv_t = tailbufs
        rows_t = sets[0][2].at[pl.ds(0, tail)]
        base_t = pl.multiple_of(ebase + n_main * CH, 8)
        isem0 = sets[0][3]
        pltpu.async_copy(src_hbm.at[pl.ds(base_t, tail)], srcv_t, isem0)
        pltpu.async_copy(dst_hbm.at[pl.ds(base_t, tail)], dstv_t, isem0)
        pltpu.make_async_copy(src_hbm.at[pl.ds(base_t, tail)], srcv_t,
                              isem0).wait()
        pltpu.make_async_copy(dst_hbm.at[pl.ds(base_t, tail)], dstv_t,
                              isem0).wait()
        pltpu.async_copy(table_hbm.at[srcv_t], rows_t, sets[0][4]).wait()
        pltpu.sync_copy(rows_t, acc_sh.at[dstv_t], add=True)
      for jj in range(SLAG, 0, -1):
        scatter_wait(sets[(n_main - jj) % NSETS])

    @pl.when(cid == 0)
    def _():
      accumulate(wa0_hbm)

    @pl.when(cid == 1)
    def _():
      accumulate(wa1_hbm)

    plsc.subcore_barrier()

    def writeback(out_hbm):
      for k in range(max_per_tile):
        j = k * ns + sid

        @pl.when(j < n_rowch)
        def _():
          off = pl.multiple_of(j * zrows, 8)
          pltpu.sync_copy(acc_sh.at[pl.ds(off, zrows)], zbuf)
          pltpu.sync_copy(zbuf, out_hbm.at[pl.ds(off, zrows)])

    @pl.when(cid == 0)
    def _():
      writeback(out0_hbm)

    @pl.when(cid == 1)
    def _():
      writeback(out1_hbm)

  return body(WA0, WA1, src, dst)


def _dense_body(x_ref, pos_ref, hA0_ref, hA1_ref, sty_ref,
                WX0x, WX0p, bX0, Wf0a, Wf0x, bf0, Wg0, Wb0, Wo0, bo0,
                WX1x, WX1p, bX1, Wf1a, Wf1x, bf1, Wg1, Wb1, Wo1, bo1,
                out_ref):
  f32 = jnp.float32
  x = x_ref[...]
  posp = pos_ref[...]
  st = sty_ref[...]

  def block(h_in, hA, WXx, WXp, bX, Wfa, Wfx, bf, Wg, Wb, Wo, bo, s):
    hX = jax.nn.relu(
        jnp.dot(h_in, WXx[...], preferred_element_type=f32)
        + jnp.dot(posp, WXp[...], preferred_element_type=f32) + bX[...])
    h = jax.nn.relu(
        jnp.dot(hA, Wfa[...], preferred_element_type=f32)
        + jnp.dot(hX, Wfx[...], preferred_element_type=f32)
        + bf[...] + hA + hX)
    gamma = 1.0 + jnp.dot(s, Wg[...], preferred_element_type=f32)
    beta = jnp.dot(s, Wb[...], preferred_element_type=f32)
    h = h * gamma + beta
    return jax.nn.relu(jnp.dot(h, Wo[...], preferred_element_type=f32)
                       + bo[...])

  s0 = (st[0:1, :] + st[1:2, :]) * 0.5
  s1 = (st[2:3, :] + st[3:4, :]) * 0.5
  h0 = block(x, hA0_ref[...], WX0x, WX0p, bX0, Wf0a, Wf0x, bf0,
             Wg0, Wb0, Wo0, bo0, s0)
  h1 = block(h0, hA1_ref[...], WX1x, WX1p, bX1, Wf1a, Wf1x, bf1,
             Wg1, Wb1, Wo1, bo1, s1)
  out_ref[:, :D] = h1
  out_ref[:, D:] = x


def _dense(x, pos8, hA0, hA1, styles_p, weights):
  BN = 2000
  grid = (N // BN,)
  row = lambda shape: pl.BlockSpec(shape, lambda i: (i, 0))
  full = lambda shape: pl.BlockSpec(shape, lambda i: (0, 0))
  in_specs = [
      row((BN, D)), row((BN, 8)), row((BN, D)), row((BN, D)),
      full((8, 64)),
  ]
  for _ in range(2):
    in_specs += [
        full((D, D)), full((8, D)), full((1, D)),
        full((D, D)), full((D, D)), full((1, D)),
        full((64, D)), full((64, D)),
        full((D, D)), full((1, D)),
    ]
  return pl.pallas_call(
      _dense_body,
      grid=grid,
      in_specs=in_specs,
      out_specs=pl.BlockSpec((BN, 2 * D), lambda i: (i, 0)),
      out_shape=jax.ShapeDtypeStruct((N, 2 * D), jnp.float32),
  )(x, pos8, hA0, hA1, styles_p, *weights)


def kernel(x, pos, edge_index, batch, styles,
           WA0, WX0, bX0, Wf0, bf0, Wg0, Wb0, Wo0, bo0,
           WA1, WX1, bX1, Wf1, bf1, Wg1, Wb1, Wo1, bo1):
  src = edge_index[0]
  dst = edge_index[1]
  hA0, hA1 = _sc_segment_sums(WA0, WA1, src, dst)

  pos8 = jnp.pad(pos, ((0, 0), (0, 5)))
  styles_p = jnp.pad(styles, ((0, 4), (0, 0)))
  weights = []
  for WX, bX, Wf, bf, Wg, Wb, Wo, bo in (
      (WX0, bX0, Wf0, bf0, Wg0, Wb0, Wo0, bo0),
      (WX1, bX1, Wf1, bf1, Wg1, Wb1, Wo1, bo1)):
    weights += [
        WX[:D], jnp.pad(WX[D:], ((0, 5), (0, 0))), bX.reshape(1, D),
        Wf[:D], Wf[D:], bf.reshape(1, D),
        Wg, Wb,
        Wo, bo.reshape(1, D),
    ]
  return _dense(x, pos8, hA0, hA1, styles_p, weights)


# speedup vs baseline: 1.0647x; 1.0647x over previous
import functools

import jax
import jax.numpy as jnp
from jax import lax
from jax.experimental import pallas as pl
from jax.experimental.pallas import tpu as pltpu
from jax.experimental.pallas import tpu_sc as plsc

N = 10000
E = 320000
D = 128
CH = 64


def _sc_segment_sums(WA0, WA1, src, dst):
  info = plsc.get_sparse_core_info()
  ns = info.num_subcores
  e_per_tile = E // ns
  n_main = e_per_tile // CH
  tail = e_per_tile - n_main * CH
  zrows = 40
  n_rowch = N // zrows
  max_per_tile = -(-n_rowch // ns)

  NSETS = 5
  GA = 3
  SLAG = 1
  IA = 4
  n_groups = n_main // NSETS

  mesh = plsc.VectorSubcoreMesh(core_axis_name="c", subcore_axis_name="s")

  scratch = []
  for _ in range(NSETS):
    scratch += [
        pltpu.VMEM((CH,), jnp.int32),
        pltpu.VMEM((CH,), jnp.int32),
        pltpu.VMEM((CH, D), jnp.float32),
        pltpu.SemaphoreType.DMA,
        pltpu.SemaphoreType.DMA,
        pltpu.SemaphoreType.DMA,
    ]
  scratch += [
      pltpu.VMEM((zrows, D), jnp.float32),
      pltpu.VMEM_SHARED((N, D), jnp.float32),
  ]
  if tail:
    scratch += [
        pltpu.VMEM((tail,), jnp.int32),
        pltpu.VMEM((tail,), jnp.int32),
    ]

  @functools.partial(
      pl.kernel,
      out_type=[
          jax.ShapeDtypeStruct((N, D), jnp.float32),
          jax.ShapeDtypeStruct((N, D), jnp.float32),
      ],
      mesh=mesh,
      scratch_types=scratch,
  )
  def body(wa0_hbm, wa1_hbm, src_hbm, dst_hbm, out0_hbm, out1_hbm, *bufs):
    sets = [bufs[6 * m:6 * m + 6] for m in range(NSETS)]
    zbuf = bufs[6 * NSETS]
    acc_sh = bufs[6 * NSETS + 1]
    tailbufs = bufs[6 * NSETS + 2:]
    cid = lax.axis_index("c")
    sid = lax.axis_index("s")
    ebase = pl.multiple_of(sid * e_per_tile, 8)

    z16 = jnp.zeros((16,), jnp.float32)

    def zrow(r, carry):
      for l in range(D // 16):
        zbuf[r, pl.ds(l * 16, 16)] = z16
      return carry

    lax.fori_loop(0, zrows, zrow, 0)
    for k in range(max_per_tile):
      j = k * ns + sid

      @pl.when(j < n_rowch)
      def _():
        pltpu.sync_copy(zbuf, acc_sh.at[pl.ds(pl.multiple_of(j * zrows, 8),
                                              zrows)])

    def idx_start(j, s):
      srcv, dstv, _, isem, _, _ = s
      base = pl.multiple_of(ebase + j * CH, 8)
      pltpu.async_copy(src_hbm.at[pl.ds(base, CH)], srcv, isem)
      pltpu.async_copy(dst_hbm.at[pl.ds(base, CH)], dstv, isem)

    def idx_wait(j, s):
      srcv, dstv, _, isem, _, _ = s
      base = pl.multiple_of(ebase + j * CH, 8)
      pltpu.make_async_copy(src_hbm.at[pl.ds(base, CH)], srcv, isem).wait()
      pltpu.make_async_copy(dst_hbm.at[pl.ds(base, CH)], dstv, isem).wait()

    for jj in range(IA):
      idx_start(jj, sets[jj])
    plsc.subcore_barrier()

    def accumulate(table_hbm):
      def gather_start(s):
        srcv, _, rows, _, gsem, _ = s
        pltpu.async_copy(table_hbm.at[srcv], rows, gsem)

      def gather_wait(s):
        srcv, _, rows, _, gsem, _ = s
        pltpu.make_async_copy(table_hbm.at[srcv], rows, gsem).wait()

      def scatter_start(s):
        _, dstv, rows, _, _, ssem = s
        pltpu.async_copy(rows, acc_sh.at[dstv], ssem, add=True)

      def scatter_wait(s):
        _, dstv, rows, _, _, ssem = s
        pltpu.make_async_copy(rows, acc_sh.at[dstv], ssem).wait()

      for jj in range(GA):
        idx_wait(jj, sets[jj])
        gather_start(sets[jj])

      def group(i, carry):
        for m in range(NSETS):
          j = i * NSETS + m
          X = sets[m]
          W = sets[(m + GA) % NSETS]
          V = sets[(m + IA) % NSETS]
          gather_wait(X)
          scatter_start(X)

          def next_gather(j=j, W=W):
            idx_wait(j + GA, W)
            gather_start(W)

          tg = (n_main - 1 - GA - m) // NSETS
          if tg >= n_groups - 1:
            next_gather()
          else:
            @pl.when(i <= tg)
            def _():
              next_gather()

          if m < SLAG:
            @pl.when(i > 0)
            def _():
              scatter_wait(V)
          else:
            scatter_wait(V)

          def next_idx(j=j, V=V):
            idx_start(j + IA, V)

          ti = (n_main - 1 - IA - m) // NSETS
          if ti >= n_groups - 1:
            next_idx()
          else:
            @pl.when(i <= ti)
            def _():
              next_idx()
        return carry

      lax.fori_loop(0, n_groups, group, 0)

      for j in range(n_groups * NSETS, n_main):
        X = sets[j % NSETS]
        gather_wait(X)
        scatter_start(X)
        scatter_wait(sets[(j - SLAG) % NSETS])

      if tail:
        srcv_t, dstv_t = tailbufs
        rows_t = sets[0][2].at[pl.ds(0, tail)]
        base_t = pl.multiple_of(ebase + n_main * CH, 8)
        isem0 = sets[0][3]
        pltpu.async_copy(src_hbm.at[pl.ds(base_t, tail)], srcv_t, isem0)
        pltpu.async_copy(dst_hbm.at[pl.ds(base_t, tail)], dstv_t, isem0)
        pltpu.make_async_copy(src_hbm.at[pl.ds(base_t, tail)], srcv_t,
                              isem0).wait()
        pltpu.make_async_copy(dst_hbm.at[pl.ds(base_t, tail)], dstv_t,
                              isem0).wait()
        pltpu.async_copy(table_hbm.at[srcv_t], rows_t, sets[0][4]).wait()
        pltpu.sync_copy(rows_t, acc_sh.at[dstv_t], add=True)
      for jj in range(SLAG, 0, -1):
        scatter_wait(sets[(n_main - jj) % NSETS])

    @pl.when(cid == 0)
    def _():
      accumulate(wa0_hbm)

    @pl.when(cid == 1)
    def _():
      accumulate(wa1_hbm)

    plsc.subcore_barrier()

    def writeback(out_hbm):
      for k in range(max_per_tile):
        j = k * ns + sid

        @pl.when(j < n_rowch)
        def _():
          off = pl.multiple_of(j * zrows, 8)
          pltpu.sync_copy(acc_sh.at[pl.ds(off, zrows)], zbuf)
          pltpu.sync_copy(zbuf, out_hbm.at[pl.ds(off, zrows)])

    @pl.when(cid == 0)
    def _():
      writeback(out0_hbm)

    @pl.when(cid == 1)
    def _():
      writeback(out1_hbm)

  return body(WA0, WA1, src, dst)


def _dense_body(x_ref, pos_ref, hA0_ref, hA1_ref, sty_ref,
                WX0x, WX0p, bX0, Wf0a, Wf0x, bf0, Wg0, Wb0, Wo0, bo0,
                WX1x, WX1p, bX1, Wf1a, Wf1x, bf1, Wg1, Wb1, Wo1, bo1,
                out_ref):
  f32 = jnp.float32
  x = x_ref[...]
  posp = pos_ref[...]
  st = sty_ref[...]

  def block(h_in, hA, WXx, WXp, bX, Wfa, Wfx, bf, Wg, Wb, Wo, bo, s):
    hX = jax.nn.relu(
        jnp.dot(h_in, WXx[...], preferred_element_type=f32)
        + jnp.dot(posp, WXp[...], preferred_element_type=f32) + bX[...])
    h = jax.nn.relu(
        jnp.dot(hA, Wfa[...], preferred_element_type=f32)
        + jnp.dot(hX, Wfx[...], preferred_element_type=f32)
        + bf[...] + hA + hX)
    gamma = 1.0 + jnp.dot(s, Wg[...], preferred_element_type=f32)
    beta = jnp.dot(s, Wb[...], preferred_element_type=f32)
    h = h * gamma + beta
    return jax.nn.relu(jnp.dot(h, Wo[...], preferred_element_type=f32)
                       + bo[...])

  s0 = (st[0:1, :] + st[1:2, :]) * 0.5
  s1 = (st[2:3, :] + st[3:4, :]) * 0.5
  h0 = block(x, hA0_ref[...], WX0x, WX0p, bX0, Wf0a, Wf0x, bf0,
             Wg0, Wb0, Wo0, bo0, s0)
  h1 = block(h0, hA1_ref[...], WX1x, WX1p, bX1, Wf1a, Wf1x, bf1,
             Wg1, Wb1, Wo1, bo1, s1)
  out_ref[:, :D] = h1
  out_ref[:, D:] = x


def _dense(x, pos8, hA0, hA1, styles_p, weights):
  BN = 2000
  grid = (N // BN,)
  row = lambda shape: pl.BlockSpec(shape, lambda i: (i, 0))
  full = lambda shape: pl.BlockSpec(shape, lambda i: (0, 0))
  in_specs = [
      row((BN, D)), row((BN, 8)), row((BN, D)), row((BN, D)),
      full((8, 64)),
  ]
  for _ in range(2):
    in_specs += [
        full((D, D)), full((8, D)), full((1, D)),
        full((D, D)), full((D, D)), full((1, D)),
        full((64, D)), full((64, D)),
        full((D, D)), full((1, D)),
    ]
  return pl.pallas_call(
      _dense_body,
      grid=grid,
      in_specs=in_specs,
      out_specs=pl.BlockSpec((BN, 2 * D), lambda i: (i, 0)),
      out_shape=jax.ShapeDtypeStruct((N, 2 * D), jnp.float32),
  )(x, pos8, hA0, hA1, styles_p, *weights)


def kernel(x, pos, edge_index, batch, styles,
           WA0, WX0, bX0, Wf0, bf0, Wg0, Wb0, Wo0, bo0,
           WA1, WX1, bX1, Wf1, bf1, Wg1, Wb1, Wo1, bo1):
  src = edge_index[0]
  dst = edge_index[1]
  hA0, hA1 = _sc_segment_sums(WA0, WA1, src, dst)

  pos8 = jnp.pad(pos, ((0, 0), (0, 5)))
  styles_p = jnp.pad(styles, ((0, 4), (0, 0)))
  weights = []
  for WX, bX, Wf, bf, Wg, Wb, Wo, bo in (
      (WX0, bX0, Wf0, bf0, Wg0, Wb0, Wo0, bo0),
      (WX1, bX1, Wf1, bf1, Wg1, Wb1, Wo1, bo1)):
    weights += [
        WX[:D], jnp.pad(WX[D:], ((0, 5), (0, 0))), bX.reshape(1, D),
        Wf[:D], Wf[D:], bf.reshape(1, D),
        Wg, Wb,
        Wo, bo.reshape(1, D),
    ]
  return _dense(x, pos8, hA0, hA1, styles_p, weights)


# speedup vs baseline: 1.0771x; 1.0117x over previous
import functools

import jax
import jax.numpy as jnp
from jax import lax
from jax.experimental import pallas as pl
from jax.experimental.pallas import tpu as pltpu
from jax.experimental.pallas import tpu_sc as plsc

N = 10000
E = 320000
D = 128
CH = 64


def _sc_segment_sums(WA0, WA1, src, dst):
  info = plsc.get_sparse_core_info()
  ns = info.num_subcores
  e_per_tile = E // ns
  n_main = e_per_tile // CH
  tail = e_per_tile - n_main * CH
  zrows = 40
  n_rowch = N // zrows
  max_per_tile = -(-n_rowch // ns)

  NSETS = 5
  GA = 3
  SLAG = 1
  IA = 4
  n_groups = n_main // NSETS

  mesh = plsc.VectorSubcoreMesh(core_axis_name="c", subcore_axis_name="s")

  scratch = []
  for _ in range(NSETS):
    scratch += [
        pltpu.VMEM((CH,), jnp.int32),
        pltpu.VMEM((CH,), jnp.int32),
        pltpu.VMEM((CH, D), jnp.float32),
        pltpu.SemaphoreType.DMA,
        pltpu.SemaphoreType.DMA,
        pltpu.SemaphoreType.DMA,
    ]
  scratch += [
      pltpu.VMEM((zrows, D), jnp.float32),
      pltpu.VMEM_SHARED((N, D), jnp.float32),
  ]
  if tail:
    scratch += [
        pltpu.VMEM((tail,), jnp.int32),
        pltpu.VMEM((tail,), jnp.int32),
    ]

  @functools.partial(
      pl.kernel,
      out_type=[
          jax.ShapeDtypeStruct((N, D), jnp.float32),
          jax.ShapeDtypeStruct((N, D), jnp.float32),
      ],
      mesh=mesh,
      scratch_types=scratch,
  )
  def body(wa0_hbm, wa1_hbm, src_hbm, dst_hbm, out0_hbm, out1_hbm, *bufs):
    sets = [bufs[6 * m:6 * m + 6] for m in range(NSETS)]
    zbuf = bufs[6 * NSETS]
    acc_sh = bufs[6 * NSETS + 1]
    tailbufs = bufs[6 * NSETS + 2:]
    cid = lax.axis_index("c")
    sid = lax.axis_index("s")
    ebase = pl.multiple_of(sid * e_per_tile, 8)

    z16 = jnp.zeros((16,), jnp.float32)

    def zrow(r, carry):
      for l in range(D // 16):
        zbuf[r, pl.ds(l * 16, 16)] = z16
      return carry

    lax.fori_loop(0, zrows, zrow, 0)
    zsem = sets[0][5]
    for k in range(max_per_tile):
      j = k * ns + sid

      @pl.when(j < n_rowch)
      def _():
        pltpu.async_copy(zbuf, acc_sh.at[pl.ds(pl.multiple_of(j * zrows, 8),
                                               zrows)], zsem)
    for k in range(max_per_tile):
      j = k * ns + sid

      @pl.when(j < n_rowch)
      def _():
        pltpu.make_async_copy(zbuf,
                              acc_sh.at[pl.ds(pl.multiple_of(j * zrows, 8),
                                              zrows)], zsem).wait()

    def idx_start(j, s):
      srcv, dstv, _, isem, _, _ = s
      base = pl.multiple_of(ebase + j * CH, 8)
      pltpu.async_copy(src_hbm.at[pl.ds(base, CH)], srcv, isem)
      pltpu.async_copy(dst_hbm.at[pl.ds(base, CH)], dstv, isem)

    def idx_wait(j, s):
      srcv, dstv, _, isem, _, _ = s
      base = pl.multiple_of(ebase + j * CH, 8)
      pltpu.make_async_copy(src_hbm.at[pl.ds(base, CH)], srcv, isem).wait()
      pltpu.make_async_copy(dst_hbm.at[pl.ds(base, CH)], dstv, isem).wait()

    for jj in range(IA):
      idx_start(jj, sets[jj])
    plsc.subcore_barrier()

    def accumulate(table_hbm):
      def gather_start(s):
        srcv, _, rows, _, gsem, _ = s
        pltpu.async_copy(table_hbm.at[srcv], rows, gsem)

      def gather_wait(s):
        srcv, _, rows, _, gsem, _ = s
        pltpu.make_async_copy(table_hbm.at[srcv], rows, gsem).wait()

      def scatter_start(s):
        _, dstv, rows, _, _, ssem = s
        pltpu.async_copy(rows, acc_sh.at[dstv], ssem, add=True)

      def scatter_wait(s):
        _, dstv, rows, _, _, ssem = s
        pltpu.make_async_copy(rows, acc_sh.at[dstv], ssem).wait()

      for jj in range(GA):
        idx_wait(jj, sets[jj])
        gather_start(sets[jj])

      def group(i, carry):
        for m in range(NSETS):
          j = i * NSETS + m
          X = sets[m]
          W = sets[(m + GA) % NSETS]
          V = sets[(m + IA) % NSETS]
          gather_wait(X)
          scatter_start(X)

          def next_gather(j=j, W=W):
            idx_wait(j + GA, W)
            gather_start(W)

          tg = (n_main - 1 - GA - m) // NSETS
          if tg >= n_groups - 1:
            next_gather()
          else:
            @pl.when(i <= tg)
            def _():
              next_gather()

          if m < SLAG:
            @pl.when(i > 0)
            def _():
              scatter_wait(V)
          else:
            scatter_wait(V)

          def next_idx(j=j, V=V):
            idx_start(j + IA, V)

          ti = (n_main - 1 - IA - m) // NSETS
          if ti >= n_groups - 1:
            next_idx()
          else:
            @pl.when(i <= ti)
            def _():
              next_idx()
        return carry

      lax.fori_loop(0, n_groups, group, 0)

      for j in range(n_groups * NSETS, n_main):
        X = sets[j % NSETS]
        gather_wait(X)
        scatter_start(X)
        scatter_wait(sets[(j - SLAG) % NSETS])

      if tail:
        srcv_t, dstv_t = tailbufs
        rows_t = sets[0][2].at[pl.ds(0, tail)]
        base_t = pl.multiple_of(ebase + n_main * CH, 8)
        isem0 = sets[0][3]
        pltpu.async_copy(src_hbm.at[pl.ds(base_t, tail)], srcv_t, isem0)
        pltpu.async_copy(dst_hbm.at[pl.ds(base_t, tail)], dstv_t, isem0)
        pltpu.make_async_copy(src_hbm.at[pl.ds(base_t, tail)], srcv_t,
                              isem0).wait()
        pltpu.make_async_copy(dst_hbm.at[pl.ds(base_t, tail)], dstv_t,
                              isem0).wait()
        pltpu.async_copy(table_hbm.at[srcv_t], rows_t, sets[0][4]).wait()
        pltpu.sync_copy(rows_t, acc_sh.at[dstv_t], add=True)
      for jj in range(SLAG, 0, -1):
        scatter_wait(sets[(n_main - jj) % NSETS])

    @pl.when(cid == 0)
    def _():
      accumulate(wa0_hbm)

    @pl.when(cid == 1)
    def _():
      accumulate(wa1_hbm)

    plsc.subcore_barrier()

    def writeback(out_hbm):
      wsem = sets[0][5]
      for k in range(max_per_tile):
        j = k * ns + sid

        @pl.when(j < n_rowch)
        def _():
          off = pl.multiple_of(j * zrows, 8)
          pltpu.async_copy(acc_sh.at[pl.ds(off, zrows)],
                           out_hbm.at[pl.ds(off, zrows)], wsem)
      for k in range(max_per_tile):
        j = k * ns + sid

        @pl.when(j < n_rowch)
        def _():
          off = pl.multiple_of(j * zrows, 8)
          pltpu.make_async_copy(acc_sh.at[pl.ds(off, zrows)],
                                out_hbm.at[pl.ds(off, zrows)], wsem).wait()

    @pl.when(cid == 0)
    def _():
      writeback(out0_hbm)

    @pl.when(cid == 1)
    def _():
      writeback(out1_hbm)

  return body(WA0, WA1, src, dst)


def _dense_body(x_ref, pos_ref, hA0_ref, hA1_ref, sty_ref,
                WX0x, WX0p, bX0, Wf0a, Wf0x, bf0, Wg0, Wb0, Wo0, bo0,
                WX1x, WX1p, bX1, Wf1a, Wf1x, bf1, Wg1, Wb1, Wo1, bo1,
                out_ref):
  f32 = jnp.float32
  x = x_ref[...]
  posp = pos_ref[...]
  st = sty_ref[...]

  def block(h_in, hA, WXx, WXp, bX, Wfa, Wfx, bf, Wg, Wb, Wo, bo, s):
    hX = jax.nn.relu(
        jnp.dot(h_in, WXx[...], preferred_element_type=f32)
        + jnp.dot(posp, WXp[...], preferred_element_type=f32) + bX[...])
    h = jax.nn.relu(
        jnp.dot(hA, Wfa[...], preferred_element_type=f32)
        + jnp.dot(hX, Wfx[...], preferred_element_type=f32)
        + bf[...] + hA + hX)
    gamma = 1.0 + jnp.dot(s, Wg[...], preferred_element_type=f32)
    beta = jnp.dot(s, Wb[...], preferred_element_type=f32)
    h = h * gamma + beta
    return jax.nn.relu(jnp.dot(h, Wo[...], preferred_element_type=f32)
                       + bo[...])

  s0 = (st[0:1, :] + st[1:2, :]) * 0.5
  s1 = (st[2:3, :] + st[3:4, :]) * 0.5
  h0 = block(x, hA0_ref[...], WX0x, WX0p, bX0, Wf0a, Wf0x, bf0,
             Wg0, Wb0, Wo0, bo0, s0)
  h1 = block(h0, hA1_ref[...], WX1x, WX1p, bX1, Wf1a, Wf1x, bf1,
             Wg1, Wb1, Wo1, bo1, s1)
  out_ref[:, :D] = h1
  out_ref[:, D:] = x


def _dense(x, pos8, hA0, hA1, styles_p, weights):
  BN = 2000
  grid = (N // BN,)
  row = lambda shape: pl.BlockSpec(shape, lambda i: (i, 0))
  full = lambda shape: pl.BlockSpec(shape, lambda i: (0, 0))
  in_specs = [
      row((BN, D)), row((BN, 8)), row((BN, D)), row((BN, D)),
      full((8, 64)),
  ]
  for _ in range(2):
    in_specs += [
        full((D, D)), full((8, D)), full((1, D)),
        full((D, D)), full((D, D)), full((1, D)),
        full((64, D)), full((64, D)),
        full((D, D)), full((1, D)),
    ]
  return pl.pallas_call(
      _dense_body,
      grid=grid,
      in_specs=in_specs,
      out_specs=pl.BlockSpec((BN, 2 * D), lambda i: (i, 0)),
      out_shape=jax.ShapeDtypeStruct((N, 2 * D), jnp.float32),
  )(x, pos8, hA0, hA1, styles_p, *weights)


def kernel(x, pos, edge_index, batch, styles,
           WA0, WX0, bX0, Wf0, bf0, Wg0, Wb0, Wo0, bo0,
           WA1, WX1, bX1, Wf1, bf1, Wg1, Wb1, Wo1, bo1):
  src = edge_index[0]
  dst = edge_index[1]
  hA0, hA1 = _sc_segment_sums(WA0, WA1, src, dst)

  pos8 = jnp.pad(pos, ((0, 0), (0, 5)))
  styles_p = jnp.pad(styles, ((0, 4), (0, 0)))
  weights = []
  for WX, bX, Wf, bf, Wg, Wb, Wo, bo in (
      (WX0, bX0, Wf0, bf0, Wg0, Wb0, Wo0, bo0),
      (WX1, bX1, Wf1, bf1, Wg1, Wb1, Wo1, bo1)):
    weights += [
        WX[:D], jnp.pad(WX[D:], ((0, 5), (0, 0))), bX.reshape(1, D),
        Wf[:D], Wf[D:], bf.reshape(1, D),
        Wg, Wb,
        Wo, bo.reshape(1, D),
    ]
  return _dense(x, pos8, hA0, hA1, styles_p, weights)


# speedup vs baseline: 1.0855x; 1.0078x over previous
import functools

import jax
import jax.numpy as jnp
from jax import lax
from jax.experimental import pallas as pl
from jax.experimental.pallas import tpu as pltpu
from jax.experimental.pallas import tpu_sc as plsc

N = 10000
E = 320000
D = 128
CH = 64


def _sc_segment_sums(WA0, WA1, src, dst):
  info = plsc.get_sparse_core_info()
  ns = info.num_subcores
  e_per_tile = E // ns
  n_main = e_per_tile // CH
  tail = e_per_tile - n_main * CH
  zrows = 40
  n_rowch = N // zrows
  max_per_tile = -(-n_rowch // ns)

  NSETS = 5
  GA = 3
  SLAG = 1
  IA = 4
  n_groups = n_main // NSETS

  mesh = plsc.VectorSubcoreMesh(core_axis_name="c", subcore_axis_name="s")

  scratch = []
  for _ in range(NSETS):
    scratch += [
        pltpu.VMEM((CH,), jnp.int32),
        pltpu.VMEM((CH,), jnp.int32),
        pltpu.VMEM((CH, D), jnp.float32),
        pltpu.SemaphoreType.DMA,
        pltpu.SemaphoreType.DMA,
        pltpu.SemaphoreType.DMA,
    ]
  scratch += [
      pltpu.VMEM((zrows, D), jnp.float32),
      pltpu.VMEM_SHARED((N, D), jnp.float32),
  ]
  if tail:
    scratch += [
        pltpu.VMEM((tail,), jnp.int32),
        pltpu.VMEM((tail,), jnp.int32),
    ]

  @functools.partial(
      pl.kernel,
      out_type=[
          jax.ShapeDtypeStruct((N, D), jnp.float32),
          jax.ShapeDtypeStruct((N, D), jnp.float32),
      ],
      mesh=mesh,
      scratch_types=scratch,
  )
  def body(wa0_hbm, wa1_hbm, src_hbm, dst_hbm, out0_hbm, out1_hbm, *bufs):
    sets = [bufs[6 * m:6 * m + 6] for m in range(NSETS)]
    zbuf = bufs[6 * NSETS]
    acc_sh = bufs[6 * NSETS + 1]
    tailbufs = bufs[6 * NSETS + 2:]
    cid = lax.axis_index("c")
    sid = lax.axis_index("s")
    ebase = pl.multiple_of(sid * e_per_tile, 8)

    z16 = jnp.zeros((16,), jnp.float32)

    def zrow(r, carry):
      for l in range(D // 16):
        zbuf[r, pl.ds(l * 16, 16)] = z16
      return carry

    lax.fori_loop(0, zrows, zrow, 0)
    zsem = sets[0][5]
    for k in range(max_per_tile):
      j = k * ns + sid

      @pl.when(j < n_rowch)
      def _():
        pltpu.async_copy(zbuf, acc_sh.at[pl.ds(pl.multiple_of(j * zrows, 8),
                                               zrows)], zsem)
    for k in range(max_per_tile):
      j = k * ns + sid

      @pl.when(j < n_rowch)
      def _():
        pltpu.make_async_copy(zbuf,
                              acc_sh.at[pl.ds(pl.multiple_of(j * zrows, 8),
                                              zrows)], zsem).wait()

    def idx_start(j, s):
      srcv, dstv, _, isem, _, _ = s
      base = pl.multiple_of(ebase + j * CH, 8)
      pltpu.async_copy(src_hbm.at[pl.ds(base, CH)], srcv, isem)
      pltpu.async_copy(dst_hbm.at[pl.ds(base, CH)], dstv, isem)

    def idx_wait(j, s):
      srcv, dstv, _, isem, _, _ = s
      base = pl.multiple_of(ebase + j * CH, 8)
      pltpu.make_async_copy(src_hbm.at[pl.ds(base, CH)], srcv, isem).wait()
      pltpu.make_async_copy(dst_hbm.at[pl.ds(base, CH)], dstv, isem).wait()

    for jj in range(IA):
      idx_start(jj, sets[jj])
    plsc.subcore_barrier()

    def accumulate(table_hbm):
      def gather_start(s):
        srcv, _, rows, _, gsem, _ = s
        pltpu.async_copy(table_hbm.at[srcv], rows, gsem)

      def gather_wait(s):
        srcv, _, rows, _, gsem, _ = s
        pltpu.make_async_copy(table_hbm.at[srcv], rows, gsem).wait()

      def scatter_start(s):
        _, dstv, rows, _, _, ssem = s
        pltpu.async_copy(rows, acc_sh.at[dstv], ssem, add=True)

      def scatter_wait(s):
        _, dstv, rows, _, _, ssem = s
        pltpu.make_async_copy(rows, acc_sh.at[dstv], ssem).wait()

      for jj in range(GA):
        idx_wait(jj, sets[jj])
        gather_start(sets[jj])

      def group(i, carry):
        for m in range(NSETS):
          j = i * NSETS + m
          X = sets[m]
          W = sets[(m + GA) % NSETS]
          V = sets[(m + IA) % NSETS]
          gather_wait(X)
          scatter_start(X)

          def next_gather(j=j, W=W):
            idx_wait(j + GA, W)
            gather_start(W)

          tg = (n_main - 1 - GA - m) // NSETS
          if tg >= n_groups - 1:
            next_gather()
          else:
            @pl.when(i <= tg)
            def _():
              next_gather()

          if m < SLAG:
            @pl.when(i > 0)
            def _():
              scatter_wait(V)
          else:
            scatter_wait(V)

          def next_idx(j=j, V=V):
            idx_start(j + IA, V)

          ti = (n_main - 1 - IA - m) // NSETS
          if ti >= n_groups - 1:
            next_idx()
          else:
            @pl.when(i <= ti)
            def _():
              next_idx()
        return carry

      lax.fori_loop(0, n_groups, group, 0)

      for j in range(n_groups * NSETS, n_main):
        X = sets[j % NSETS]
        gather_wait(X)
        scatter_start(X)
        scatter_wait(sets[(j - SLAG) % NSETS])

      if tail:
        srcv_t, dstv_t = tailbufs
        rows_t = sets[0][2].at[pl.ds(0, tail)]
        base_t = pl.multiple_of(ebase + n_main * CH, 8)
        isem0 = sets[0][3]
        pltpu.async_copy(src_hbm.at[pl.ds(base_t, tail)], srcv_t, isem0)
        pltpu.async_copy(dst_hbm.at[pl.ds(base_t, tail)], dstv_t, isem0)
        pltpu.make_async_copy(src_hbm.at[pl.ds(base_t, tail)], srcv_t,
                              isem0).wait()
        pltpu.make_async_copy(dst_hbm.at[pl.ds(base_t, tail)], dstv_t,
                              isem0).wait()
        pltpu.async_copy(table_hbm.at[srcv_t], rows_t, sets[0][4]).wait()
        pltpu.sync_copy(rows_t, acc_sh.at[dstv_t], add=True)
      for jj in range(SLAG, 0, -1):
        scatter_wait(sets[(n_main - jj) % NSETS])

    @pl.when(cid == 0)
    def _():
      accumulate(wa0_hbm)

    @pl.when(cid == 1)
    def _():
      accumulate(wa1_hbm)

    plsc.subcore_barrier()

    def writeback(out_hbm):
      wsem = sets[0][5]
      for k in range(max_per_tile):
        j = k * ns + sid

        @pl.when(j < n_rowch)
        def _():
          off = pl.multiple_of(j * zrows, 8)
          pltpu.async_copy(acc_sh.at[pl.ds(off, zrows)],
                           out_hbm.at[pl.ds(off, zrows)], wsem)
      for k in range(max_per_tile):
        j = k * ns + sid

        @pl.when(j < n_rowch)
        def _():
          off = pl.multiple_of(j * zrows, 8)
          pltpu.make_async_copy(acc_sh.at[pl.ds(off, zrows)],
                                out_hbm.at[pl.ds(off, zrows)], wsem).wait()

    @pl.when(cid == 0)
    def _():
      writeback(out0_hbm)

    @pl.when(cid == 1)
    def _():
      writeback(out1_hbm)

  return body(WA0, WA1, src, dst)


def _dense_body(x_ref, pos_ref, hA0_ref, hA1_ref, sty_ref,
                WX0, bX0, Wf0, bf0, Wg0, Wb0, Wo0, bo0,
                WX1, bX1, Wf1, bf1, Wg1, Wb1, Wo1, bo1,
                out_ref):
  f32 = jnp.float32
  x = x_ref[...]
  xp = jnp.concatenate([x, pos_ref[...]], axis=1)
  st = sty_ref[...]

  def block(hin_p, hA, WX, bX, Wf, bf, Wg, Wb, Wo, bo, s):
    hX = jax.nn.relu(jnp.dot(hin_p, WX[...], preferred_element_type=f32)
                     + bX[...][None, :])
    hcat = jnp.concatenate([hA, hX], axis=1)
    h = jax.nn.relu(jnp.dot(hcat, Wf[...], preferred_element_type=f32)
                    + bf[...][None, :] + hA + hX)
    gamma = 1.0 + jnp.dot(s, Wg[...], preferred_element_type=f32)
    beta = jnp.dot(s, Wb[...], preferred_element_type=f32)
    h = h * gamma + beta
    return jax.nn.relu(jnp.dot(h, Wo[...], preferred_element_type=f32)
                       + bo[...][None, :])

  s0 = (st[0:1, :] + st[1:2, :]) * 0.5
  s1 = (st[2:3, :] + st[3:4, :]) * 0.5
  h0 = block(xp, hA0_ref[...], WX0, bX0, Wf0, bf0, Wg0, Wb0, Wo0, bo0, s0)
  hp = jnp.concatenate([h0, pos_ref[...]], axis=1)
  h1 = block(hp, hA1_ref[...], WX1, bX1, Wf1, bf1, Wg1, Wb1, Wo1, bo1, s1)
  out_ref[:, :D] = h1
  out_ref[:, D:] = x


def _dense(x, pos, hA0, hA1, styles, weights):
  BN = 2000
  grid = (N // BN,)
  row = lambda shape: pl.BlockSpec(shape, lambda i: (i, 0))
  full2 = lambda shape: pl.BlockSpec(shape, lambda i: (0, 0))
  full1 = lambda shape: pl.BlockSpec(shape, lambda i: (0,))
  in_specs = [
      row((BN, D)), row((BN, 3)), row((BN, D)), row((BN, D)),
      full2((4, 64)),
  ]
  for _ in range(2):
    in_specs += [
        full2((D + 3, D)), full1((D,)),
        full2((2 * D, D)), full1((D,)),
        full2((64, D)), full2((64, D)),
        full2((D, D)), full1((D,)),
    ]
  return pl.pallas_call(
      _dense_body,
      grid=grid,
      in_specs=in_specs,
      out_specs=pl.BlockSpec((BN, 2 * D), lambda i: (i, 0)),
      out_shape=jax.ShapeDtypeStruct((N, 2 * D), jnp.float32),
  )(x, pos, hA0, hA1, styles, *weights)


def kernel(x, pos, edge_index, batch, styles,
           WA0, WX0, bX0, Wf0, bf0, Wg0, Wb0, Wo0, bo0,
           WA1, WX1, bX1, Wf1, bf1, Wg1, Wb1, Wo1, bo1):
  src = edge_index[0]
  dst = edge_index[1]
  hA0, hA1 = _sc_segment_sums(WA0, WA1, src, dst)
  weights = [WX0, bX0, Wf0, bf0, Wg0, Wb0, Wo0, bo0,
             WX1, bX1, Wf1, bf1, Wg1, Wb1, Wo1, bo1]
  return _dense(x, pos, hA0, hA1, styles, weights)
